# baseline (speedup 1.0000x reference)
import functools

import jax
import jax.numpy as jnp
from jax import lax
from jax.experimental import pallas as pl
from jax.experimental.pallas import tpu as pltpu
from jax.experimental.pallas import tpu_sc as plsc

SEQ = 200
BATCH = 1024
DIM = 64
B = SEQ * BATCH
NC = 2
NS = 16
NW = NC * NS
BPW = B // NW
CHUNK = 128
NCH = BPW // CHUNK
NBUF = 8
CPB = BATCH // CHUNK

_mesh = plsc.VectorSubcoreMesh(core_axis_name="c", subcore_axis_name="s")


@functools.partial(
    pl.kernel,
    mesh=_mesh,
    compiler_params=pltpu.CompilerParams(use_tc_tiling_on_sc=False),
    out_type=jax.ShapeDtypeStruct((SEQ, BATCH, DIM), jnp.float32),
    scratch_types=[
        pltpu.VMEM((BPW,), jnp.int32),
        pltpu.VMEM((NBUF, CHUNK, DIM), jnp.float32),
        pltpu.SemaphoreType.DMA,
        pltpu.SemaphoreType.DMA,
    ],
)
def _embed_lookup(idx_hbm, table_hbm, out_hbm, idx_v, rows_v, gsem, psem):
    wid = lax.axis_index("s") * NC + lax.axis_index("c")
    base = pl.multiple_of(wid * BPW, BPW)
    pltpu.sync_copy(idx_hbm.at[pl.ds(base, BPW)], idx_v)
    cbase = wid * NCH

    def out_ref(g):
        return out_hbm.at[lax.div(g, CPB), pl.ds(lax.rem(g, CPB) * CHUNK, CHUNK)]

    def idx_ref(j):
        return idx_v.at[pl.ds(pl.multiple_of(j * CHUNK, CHUNK), CHUNK)]

    for j in range(NBUF):
        pltpu.async_copy(table_hbm.at[idx_ref(j)], rows_v.at[j], gsem)

    @pl.loop(0, NCH)
    def _chunk(j):
        slot = lax.rem(j, NBUF)
        pltpu.make_async_copy(table_hbm.at[idx_ref(j)], rows_v.at[slot], gsem).wait()
        dst = out_ref(cbase + j)
        pltpu.async_copy(rows_v.at[slot], dst, psem)
        pltpu.make_async_copy(rows_v.at[slot], dst, psem).wait()

        @pl.when(j + NBUF < NCH)
        def _():
            pltpu.async_copy(table_hbm.at[idx_ref(j + NBUF)], rows_v.at[slot], gsem)


def kernel(source, table):
    idx = source.reshape(B)
    return _embed_lookup(idx, table)
